# baseline (speedup 1.0000x reference)
import jax
import jax.numpy as jnp
from jax import lax
from jax.experimental import pallas as pl
from jax.experimental.pallas import tpu as pltpu
from jax.experimental.pallas import tpu_sc as plsc

BATCH = 16384
ED = 64
L = 16
NUM_CORES = 2
NUM_SUBCORES = 16
NW = NUM_CORES * NUM_SUBCORES

NUSERS = 1_000_000
NGAMES = 100_000
NC_U = (NUSERS + 127) // 128
NC_G = (NGAMES + 127) // 128
CPT_U = (NC_U + NW - 1) // NW
CPT_G = (NC_G + NW - 1) // NW
WCOLS = 16
WROWS = WCOLS * 128
NB_U = (CPT_U + WCOLS - 1) // WCOLS
NB_G = (CPT_G + WCOLS - 1) // WCOLS
MCAP = 768
BCAP = MCAP + NB_U * L + L
JROW = BATCH
OROWS = BATCH + 128
OFLAT = OROWS * ED
SUBBLK = 4096
NGRP = BATCH // L
B_PER_W = BATCH // NW


def _scan_pass(tbl_hbm, ids_hbm, out_hbm, nc, cpt, nb, shift_nb,
               idbuf, ulist, ilist, u2, i2, mflat, slab_a, slab_b,
               cbuf, pbuf, maskbuf, smem_off, sem_a, sem_b, sem_s, wid):
  base_col = wid * cpt
  ubase = base_col * 128
  utop = (base_col + cpt) * 128
  lane_iota = lax.iota(jnp.int32, L)
  nslab = 8 * nb

  def p1blk(blk, carry):
    pltpu.sync_copy(ids_hbm.at[pl.ds(blk * SUBBLK, SUBBLK)], idbuf)

    def p1sg(sg, c):
      for gg in range(L):
        v = idbuf[pl.ds(sg * 256 + gg * L, L)]
        m = (v >= ubase) & (v < utop)
        maskbuf[pl.ds(gg * L, L)] = m.astype(jnp.int32)
      cnt = None
      for col in range(L):
        g = plsc.load_gather(maskbuf, [lane_iota * L + col])
        cnt = g if cnt is None else cnt + g
      cbuf[pl.ds(blk * 256 + sg * L, L)] = cnt
      return c

    return lax.fori_loop(0, SUBBLK // 256, p1sg, carry)

  lax.fori_loop(0, BATCH // SUBBLK, p1blk, None)

  def pfx(ch, base):
    cv = cbuf[pl.ds(ch * L, L)]
    inc = plsc.cumsum(cv)
    pbuf[pl.ds(ch * L, L)] = base + (inc - cv)
    return base + inc[L - 1]

  total = lax.fori_loop(0, NGRP // L, pfx, jnp.int32(0))
  nwaves = (total + MCAP - 1) // MCAP

  def slab_col(s):
    b = s & (nb - 1)
    return jnp.minimum(base_col + b * WCOLS, nc - WCOLS)

  def issue(s, buf, sem):
    tf = s >> shift_nb
    row0 = pl.multiple_of(tf * 8, 8)
    col0 = pl.multiple_of(slab_col(s) * 128, 128)
    pltpu.async_copy(tbl_hbm.at[pl.ds(row0, 8), pl.ds(col0, WROWS)], buf, sem)

  def wait_slab(buf, sem):
    pltpu.make_async_copy(
        tbl_hbm.at[pl.ds(0, 8), pl.ds(0, WROWS)], buf, sem).wait()

  def process(s, buf):
    tf = s >> shift_nb
    b = s & (nb - 1)
    sbase = slab_col(s) * 128
    j0 = smem_off[b] >> 4
    j1 = smem_off[b + 1] >> 4

    def pg(j16, carry):
      jv = j16 * L + lane_iota
      uv = u2[pl.ds(j16 * L, L)]
      x = uv - sbase
      for fo in range(8):
        val = plsc.load_gather(buf, [jnp.full((L,), fo, jnp.int32), x])
        plsc.store_scatter(mflat, [jv * ED + (tf * 8 + fo)], val)
      return carry

    lax.fori_loop(j0, j1, pg, None)

  def wave(vw, carry0):
    lo = vw * MCAP
    hi = lo + MCAP

    issue(0, slab_a, sem_a)

    def blk_loop(blk, carry):
      pltpu.sync_copy(ids_hbm.at[pl.ds(blk * SUBBLK, SUBBLK)], idbuf)

      def g_loop(g, c):
        gi = blk * (SUBBLK // L) + g
        v = idbuf[pl.ds(g * L, L)]
        m = (v >= ubase) & (v < utop)
        mi = m.astype(jnp.int32)
        base = pbuf[pl.ds(gi, L)][0]
        rank = base + plsc.cumsum(mi) - mi
        sel = m & (rank >= lo) & (rank < hi)
        off = jnp.clip(base - lo, 0, MCAP)
        plsc.store_compressed(ulist.at[pl.ds(off, L)], v, mask=sel)
        ivec = blk * SUBBLK + g * L + lane_iota
        plsc.store_compressed(ilist.at[pl.ds(off, L)], ivec, mask=sel)
        return c

      return lax.fori_loop(0, SUBBLK // L, g_loop, carry)

    lax.fori_loop(0, BATCH // SUBBLK, blk_loop, None)
    moff = jnp.clip(total - lo, 0, MCAP)
    ulist[pl.ds(moff, L)] = jnp.full((L,), -1, jnp.int32)
    ilist[pl.ds(moff, L)] = jnp.full((L,), JROW, jnp.int32)
    ng = (moff + L - 1) >> 4

    seg = jnp.int32(0)
    for b in range(nb):
      bb = ubase + b * WROWS
      bt = bb + WROWS
      smem_off[b] = seg

      def bg(g, c, bb=bb, bt=bt):
        v = ulist[pl.ds(g * L, L)]
        m = (v >= bb) & (v < bt)
        cnt = plsc.all_reduce_population_count(m)[0]
        plsc.store_compressed(u2.at[pl.ds(c, L)], v, mask=m)
        iv = ilist[pl.ds(g * L, L)]
        plsc.store_compressed(i2.at[pl.ds(c, L)], iv, mask=m)
        return c + cnt

      seg = lax.fori_loop(0, ng, bg, seg)
      u2[pl.ds(seg, L)] = jnp.full((L,), bb, jnp.int32)
      i2[pl.ds(seg, L)] = jnp.full((L,), JROW, jnp.int32)
      seg = (seg + L - 1) & ~(L - 1)
    smem_off[nb] = seg

    def ring(q, carry):
      s0 = 2 * q
      s1 = s0 + 1
      issue(s1, slab_b, sem_b)
      wait_slab(slab_a, sem_a)
      process(s0, slab_a)
      issue(jnp.minimum(s1 + 1, nslab - 1), slab_a, sem_a)
      wait_slab(slab_b, sem_b)
      process(s1, slab_b)
      return carry

    lax.fori_loop(0, nslab // 2, ring, None)
    wait_slab(slab_a, sem_a)

    jtot = smem_off[nb]

    def sc16(j16, carry):
      iv = i2[pl.ds(j16 * L, L)]
      for k in range(L):
        srco = pl.multiple_of((j16 * L + k) * ED, 8)
        dsto = pl.multiple_of(iv[k] * ED, 8)
        pltpu.async_copy(mflat.at[pl.ds(srco, ED)],
                         out_hbm.at[pl.ds(dsto, ED)], sem_s)
      return carry

    lax.fori_loop(0, jtot >> 4, sc16, None)

    def scd(j, carry):
      pltpu.make_async_copy(mflat.at[pl.ds(0, ED)],
                            out_hbm.at[pl.ds(0, ED)], sem_s).wait()
      return carry

    lax.fori_loop(0, jtot, scd, None)
    return carry0

  lax.fori_loop(0, nwaves, wave, None)


def _route_body(user_ids_hbm, game_ids_hbm, ut_hbm, gt_hbm,
                ug_hbm, gg_hbm, idbuf, ulist, ilist, u2, i2, mflat,
                slab_a, slab_b, cbuf, pbuf, maskbuf,
                smem_off, sem_a, sem_b, sem_s):
  wid = lax.axis_index("s") * NUM_CORES + lax.axis_index("c")
  scratch = (idbuf, ulist, ilist, u2, i2, mflat, slab_a, slab_b,
             cbuf, pbuf, maskbuf, smem_off, sem_a, sem_b, sem_s)
  _scan_pass(ut_hbm, user_ids_hbm, ug_hbm, NC_U, CPT_U, NB_U, 4,
             *scratch, wid)
  _scan_pass(gt_hbm, game_ids_hbm, gg_hbm, NC_G, CPT_G, NB_G, 1,
             *scratch, wid)


def _dot_body(ug_hbm, gg_hbm, out_hbm, ubuf, gbuf, out_v, acc_buf, sem):
  wid = lax.axis_index("s") * NUM_CORES + lax.axis_index("c")
  base = wid * B_PER_W
  pltpu.async_copy(ug_hbm.at[pl.ds(base * ED, B_PER_W * ED)], ubuf, sem)
  pltpu.async_copy(gg_hbm.at[pl.ds(base * ED, B_PER_W * ED)], gbuf, sem)
  pltpu.make_async_copy(ug_hbm.at[pl.ds(0, B_PER_W * ED)], ubuf, sem).wait()
  pltpu.make_async_copy(gg_hbm.at[pl.ds(0, B_PER_W * ED)], gbuf, sem).wait()

  lane_iota = lax.iota(jnp.int32, L)

  def group(t, carry):
    for k in range(L):
      r = t * L + k
      acc = None
      for j in range(ED // L):
        u_j = ubuf[pl.ds(r * ED + j * L, L)]
        g_j = gbuf[pl.ds(r * ED + j * L, L)]
        p = u_j * g_j
        acc = p if acc is None else acc + p
      acc_buf[pl.ds(k * L, L)] = acc
    tot = None
    for c in range(L):
      v = plsc.load_gather(acc_buf, [lane_iota * L + c])
      tot = v if tot is None else tot + v
    out_v[pl.ds(t * L, L)] = tot
    return carry

  lax.fori_loop(0, B_PER_W // L, group, None)
  pltpu.sync_copy(out_v, out_hbm.at[pl.ds(base, B_PER_W)])


@jax.jit
def kernel(user_ids, game_ids, user_table, game_table):
  mesh = plsc.VectorSubcoreMesh(core_axis_name="c", subcore_axis_name="s")
  params = pltpu.CompilerParams(needs_layout_passes=False)

  route = pl.kernel(
      _route_body,
      out_type=(jax.ShapeDtypeStruct((OFLAT,), jnp.float32),
                jax.ShapeDtypeStruct((OFLAT,), jnp.float32)),
      mesh=mesh,
      scratch_types=[
          pltpu.VMEM((SUBBLK,), jnp.int32),
          pltpu.VMEM((MCAP + L,), jnp.int32),
          pltpu.VMEM((MCAP + L,), jnp.int32),
          pltpu.VMEM((BCAP + L,), jnp.int32),
          pltpu.VMEM((BCAP + L,), jnp.int32),
          pltpu.VMEM((BCAP * ED,), jnp.float32),
          pltpu.VMEM((8, WROWS), jnp.float32),
          pltpu.VMEM((8, WROWS), jnp.float32),
          pltpu.VMEM((NGRP + L,), jnp.int32),
          pltpu.VMEM((NGRP + L,), jnp.int32),
          pltpu.VMEM((256,), jnp.int32),
          pltpu.SMEM((NB_U + 1,), jnp.int32),
          pltpu.SemaphoreType.DMA,
          pltpu.SemaphoreType.DMA,
          pltpu.SemaphoreType.DMA,
      ],
      compiler_params=params,
  )
  ug, gg = route(user_ids, game_ids, user_table.T, game_table.T)

  dot = pl.kernel(
      _dot_body,
      out_type=jax.ShapeDtypeStruct((BATCH,), jnp.float32),
      mesh=mesh,
      scratch_types=[
          pltpu.VMEM((B_PER_W * ED,), jnp.float32),
          pltpu.VMEM((B_PER_W * ED,), jnp.float32),
          pltpu.VMEM((B_PER_W,), jnp.float32),
          pltpu.VMEM((L * L,), jnp.float32),
          pltpu.SemaphoreType.DMA,
      ],
      compiler_params=params,
  )
  return dot(ug, gg)


# speedup vs baseline: 1.0250x; 1.0250x over previous
import jax
import jax.numpy as jnp
from jax import lax
from jax.experimental import pallas as pl
from jax.experimental.pallas import tpu as pltpu
from jax.experimental.pallas import tpu_sc as plsc

BATCH = 16384
ED = 64
L = 16
NUM_CORES = 2
NUM_SUBCORES = 16
NW = NUM_CORES * NUM_SUBCORES

NUSERS = 1_000_000
NGAMES = 100_000
NC_U = (NUSERS + 127) // 128
NC_G = (NGAMES + 127) // 128
CPT_U = (NC_U + NW - 1) // NW
CPT_G = (NC_G + NW - 1) // NW
WCOLS = 16
WROWS = WCOLS * 128
NB_U = (CPT_U + WCOLS - 1) // WCOLS
NB_G = (CPT_G + WCOLS - 1) // WCOLS
MCAP = 768
BCAP = MCAP + NB_U * L + L
JROW = BATCH
OROWS = BATCH + 128
OFLAT = OROWS * ED
SUBBLK = 4096
B_PER_W = BATCH // NW


def _scan_pass(tbl_hbm, ids_hbm, out_hbm, nc, cpt, nb, shift_nb,
               idbuf, ulist, ilist, u2, i2, mflat, slab_a, slab_b,
               smem_off, sem_a, sem_b, sem_s, wid):
  base_col = wid * cpt
  ubase = base_col * 128
  utop = (base_col + cpt) * 128
  lane_iota = lax.iota(jnp.int32, L)
  nslab = 8 * nb

  def cblk(blk, tot):
    pltpu.sync_copy(ids_hbm.at[pl.ds(blk * SUBBLK, SUBBLK)], idbuf)

    def cg(g, t):
      v = idbuf[pl.ds(g * L, L)]
      m = (v >= ubase) & (v < utop)
      return t + jnp.sum(m.astype(jnp.int32))

    return lax.fori_loop(0, SUBBLK // L, cg, tot)

  total = lax.fori_loop(0, BATCH // SUBBLK, cblk, jnp.int32(0))
  nwaves = (total + MCAP - 1) // MCAP

  def slab_col(s):
    b = s & (nb - 1)
    return jnp.minimum(base_col + b * WCOLS, nc - WCOLS)

  def issue(s, buf, sem):
    tf = s >> shift_nb
    row0 = pl.multiple_of(tf * 8, 8)
    col0 = pl.multiple_of(slab_col(s) * 128, 128)
    pltpu.async_copy(tbl_hbm.at[pl.ds(row0, 8), pl.ds(col0, WROWS)], buf, sem)

  def wait_slab(buf, sem):
    pltpu.make_async_copy(
        tbl_hbm.at[pl.ds(0, 8), pl.ds(0, WROWS)], buf, sem).wait()

  def process(s, buf):
    tf = s >> shift_nb
    b = s & (nb - 1)
    sbase = slab_col(s) * 128
    j0 = smem_off[b] >> 4
    j1 = smem_off[b + 1] >> 4

    def pg(j16, carry):
      jv = j16 * L + lane_iota
      uv = u2[pl.ds(j16 * L, L)]
      x = uv - sbase
      for fo in range(8):
        val = plsc.load_gather(buf, [jnp.full((L,), fo, jnp.int32), x])
        plsc.store_scatter(mflat, [jv * ED + (tf * 8 + fo)], val)
      return carry

    lax.fori_loop(j0, j1, pg, None)

  def wave(vw, carry0):
    lo = vw * MCAP
    hi = lo + MCAP

    def blk_loop(blk, carry):
      pltpu.sync_copy(ids_hbm.at[pl.ds(blk * SUBBLK, SUBBLK)], idbuf)

      def g_loop(g, c):
        off, grank = c
        v = idbuf[pl.ds(g * L, L)]
        m = (v >= ubase) & (v < utop)
        mi = m.astype(jnp.int32)
        rank = grank + plsc.cumsum(mi) - mi
        sel = m & (rank >= lo) & (rank < hi)
        cnt = jnp.sum(sel.astype(jnp.int32))
        plsc.store_compressed(ulist.at[pl.ds(off, L)], v, mask=sel)
        ivec = blk * SUBBLK + g * L + lane_iota
        plsc.store_compressed(ilist.at[pl.ds(off, L)], ivec, mask=sel)
        return off + cnt, grank + jnp.sum(mi)

      return lax.fori_loop(0, SUBBLK // L, g_loop, carry)

    moff, _ = lax.fori_loop(0, BATCH // SUBBLK, blk_loop,
                            (jnp.int32(0), jnp.int32(0)))
    ulist[pl.ds(moff, L)] = jnp.full((L,), -1, jnp.int32)
    ilist[pl.ds(moff, L)] = jnp.full((L,), JROW, jnp.int32)
    ng = (moff + L - 1) >> 4

    seg = jnp.int32(0)
    for b in range(nb):
      bb = ubase + b * WROWS
      bt = bb + WROWS
      smem_off[b] = seg

      def bg(g, c, bb=bb, bt=bt):
        v = ulist[pl.ds(g * L, L)]
        m = (v >= bb) & (v < bt)
        cnt = jnp.sum(m.astype(jnp.int32))
        plsc.store_compressed(u2.at[pl.ds(c, L)], v, mask=m)
        iv = ilist[pl.ds(g * L, L)]
        plsc.store_compressed(i2.at[pl.ds(c, L)], iv, mask=m)
        return c + cnt

      seg = lax.fori_loop(0, ng, bg, seg)
      u2[pl.ds(seg, L)] = jnp.full((L,), bb, jnp.int32)
      i2[pl.ds(seg, L)] = jnp.full((L,), JROW, jnp.int32)
      seg = (seg + L - 1) & ~(L - 1)
    smem_off[nb] = seg

    issue(0, slab_a, sem_a)

    def ring(q, carry):
      s0 = 2 * q
      s1 = s0 + 1
      issue(s1, slab_b, sem_b)
      wait_slab(slab_a, sem_a)
      process(s0, slab_a)
      issue(jnp.minimum(s1 + 1, nslab - 1), slab_a, sem_a)
      wait_slab(slab_b, sem_b)
      process(s1, slab_b)
      return carry

    lax.fori_loop(0, nslab // 2, ring, None)
    wait_slab(slab_a, sem_a)

    jtot = smem_off[nb]

    def sc(j, carry):
      i = i2[pl.ds(j, L)][0]
      src = pl.multiple_of(j * ED, 8)
      dst = pl.multiple_of(i * ED, 8)
      pltpu.async_copy(mflat.at[pl.ds(src, ED)],
                       out_hbm.at[pl.ds(dst, ED)], sem_s)
      return carry

    lax.fori_loop(0, jtot, sc, None)

    def scd(j, carry):
      pltpu.make_async_copy(mflat.at[pl.ds(0, ED)],
                            out_hbm.at[pl.ds(0, ED)], sem_s).wait()
      return carry

    lax.fori_loop(0, jtot, scd, None)
    return carry0

  lax.fori_loop(0, nwaves, wave, None)


def _route_body(user_ids_hbm, game_ids_hbm, ut_hbm, gt_hbm,
                ug_hbm, gg_hbm, idbuf, ulist, ilist, u2, i2, mflat,
                slab_a, slab_b, smem_off, sem_a, sem_b, sem_s):
  wid = lax.axis_index("s") * NUM_CORES + lax.axis_index("c")
  scratch = (idbuf, ulist, ilist, u2, i2, mflat, slab_a, slab_b,
             smem_off, sem_a, sem_b, sem_s)
  _scan_pass(ut_hbm, user_ids_hbm, ug_hbm, NC_U, CPT_U, NB_U, 4,
             *scratch, wid)
  _scan_pass(gt_hbm, game_ids_hbm, gg_hbm, NC_G, CPT_G, NB_G, 1,
             *scratch, wid)


def _dot_body(ug_hbm, gg_hbm, out_hbm, ubuf, gbuf, out_v, acc_buf, sem):
  wid = lax.axis_index("s") * NUM_CORES + lax.axis_index("c")
  base = wid * B_PER_W
  pltpu.async_copy(ug_hbm.at[pl.ds(base * ED, B_PER_W * ED)], ubuf, sem)
  pltpu.async_copy(gg_hbm.at[pl.ds(base * ED, B_PER_W * ED)], gbuf, sem)
  pltpu.make_async_copy(ug_hbm.at[pl.ds(0, B_PER_W * ED)], ubuf, sem).wait()
  pltpu.make_async_copy(gg_hbm.at[pl.ds(0, B_PER_W * ED)], gbuf, sem).wait()

  lane_iota = lax.iota(jnp.int32, L)

  def group(t, carry):
    for k in range(L):
      r = t * L + k
      acc = None
      for j in range(ED // L):
        u_j = ubuf[pl.ds(r * ED + j * L, L)]
        g_j = gbuf[pl.ds(r * ED + j * L, L)]
        p = u_j * g_j
        acc = p if acc is None else acc + p
      acc_buf[pl.ds(k * L, L)] = acc
    tot = None
    for c in range(L):
      v = plsc.load_gather(acc_buf, [lane_iota * L + c])
      tot = v if tot is None else tot + v
    out_v[pl.ds(t * L, L)] = tot
    return carry

  lax.fori_loop(0, B_PER_W // L, group, None)
  pltpu.sync_copy(out_v, out_hbm.at[pl.ds(base, B_PER_W)])


@jax.jit
def kernel(user_ids, game_ids, user_table, game_table):
  mesh = plsc.VectorSubcoreMesh(core_axis_name="c", subcore_axis_name="s")
  params = pltpu.CompilerParams(needs_layout_passes=False)

  route = pl.kernel(
      _route_body,
      out_type=(jax.ShapeDtypeStruct((OFLAT,), jnp.float32),
                jax.ShapeDtypeStruct((OFLAT,), jnp.float32)),
      mesh=mesh,
      scratch_types=[
          pltpu.VMEM((SUBBLK,), jnp.int32),
          pltpu.VMEM((MCAP + L,), jnp.int32),
          pltpu.VMEM((MCAP + L,), jnp.int32),
          pltpu.VMEM((BCAP + L,), jnp.int32),
          pltpu.VMEM((BCAP + L,), jnp.int32),
          pltpu.VMEM((BCAP * ED,), jnp.float32),
          pltpu.VMEM((8, WROWS), jnp.float32),
          pltpu.VMEM((8, WROWS), jnp.float32),
          pltpu.SMEM((NB_U + 1,), jnp.int32),
          pltpu.SemaphoreType.DMA,
          pltpu.SemaphoreType.DMA,
          pltpu.SemaphoreType.DMA,
      ],
      compiler_params=params,
  )
  ug, gg = route(user_ids, game_ids, user_table.T, game_table.T)

  dot = pl.kernel(
      _dot_body,
      out_type=jax.ShapeDtypeStruct((BATCH,), jnp.float32),
      mesh=mesh,
      scratch_types=[
          pltpu.VMEM((B_PER_W * ED,), jnp.float32),
          pltpu.VMEM((B_PER_W * ED,), jnp.float32),
          pltpu.VMEM((B_PER_W,), jnp.float32),
          pltpu.VMEM((L * L,), jnp.float32),
          pltpu.SemaphoreType.DMA,
      ],
      compiler_params=params,
  )
  return dot(ug, gg)
